# Initial kernel scaffold; baseline (speedup 1.0000x reference)
import jax
import jax.numpy as jnp
from jax.experimental import pallas as pl


def kernel(y_pred, y, data):
    raise NotImplementedError("write your pallas kernel here")


# speedup vs baseline: 8.5140x; 8.5140x over previous
import jax
import jax.numpy as jnp
from jax import lax
from jax.experimental import pallas as pl
from jax.experimental.pallas import tpu as pltpu
from jax.experimental.pallas import tpu_sc as plsc

_B = 16384
_E = 210
_W = 2 * _E
_NC = 2
_NS = 16
_NW = _NC * _NS
_RPT = _B // _NW
_RB = 16
_NB = _RPT // _RB
_HI = 222.0
_INV700 = 1.0 / 700.0


def _sc_body(yp_hbm, y_hbm, tab_hbm, out_hbm, tab_v, bufp, bufy, tmp_v):
    wid = lax.axis_index("s") * _NC + lax.axis_index("c")
    pltpu.sync_copy(tab_hbm, tab_v)
    lanes = lax.iota(jnp.int32, 16)
    lane_base = lanes * _W
    zeros = jnp.zeros((16,), jnp.float32)
    base_word = wid * _RPT * _W

    def block_body(b, carry):
        zn, ir, sq = carry
        w0 = base_word + b * (_RB * _W)
        pltpu.sync_copy(yp_hbm.at[pl.ds(w0, _RB * _W)], bufp)
        pltpu.sync_copy(y_hbm.at[pl.ds(w0, _RB * _W)], bufy)

        def elem_body(e, c):
            zn, ir, ct, cp = c
            cid = lane_base + 2 * e
            x = plsc.load_gather(bufp, [cid])
            a = plsc.load_gather(bufp, [cid + 1])
            ti = plsc.load_gather(bufy, [cid])
            ta = plsc.load_gather(bufy, [cid + 1])
            p = jnp.exp(x * -8.0)
            q = jnp.exp(a * -8.0)
            zn = zn + (2.0 * (p + q) - 4.0 * (p * q)) / ((1.0 + p) * (1.0 + q))
            ir = ir + jnp.maximum(x - _HI, 0.0)
            valid = (x > 0.0) & (x <= _HI)
            xi = jnp.where(valid, x, 0.0).astype(jnp.int32)
            cp = cp + plsc.load_gather(tab_v, [xi]) * a
            ct = ct + plsc.load_gather(tab_v, [ti.astype(jnp.int32)]) * ta
            return zn, ir, ct, cp

        zn, ir, ct, cp = lax.fori_loop(0, _E, elem_body, (zn, ir, zeros, zeros))
        d = (ct - cp) * _INV700
        return zn, ir, sq + d * d

    zn, ir, sq = lax.fori_loop(0, _NB, block_body, (zeros, zeros, zeros))
    znS = jnp.sum(zn)
    irS = jnp.sum(ir)
    sqS = jnp.sum(sq)
    outv = jnp.where(lanes == 0, znS,
                     jnp.where(lanes == 1, irS,
                               jnp.where(lanes == 2, sqS, 0.0)))
    tmp_v[...] = outv
    pltpu.sync_copy(tmp_v, out_hbm.at[wid])


def kernel(y_pred, y, data):
    ypr = y_pred.reshape(_B * _W)
    yr = y.reshape(_B * _W)
    tab = jnp.concatenate([data[:, 0], jnp.zeros((1,), jnp.float32)])
    call = pl.kernel(
        _sc_body,
        out_type=jax.ShapeDtypeStruct((_NW, 16), jnp.float32),
        mesh=plsc.VectorSubcoreMesh(core_axis_name="c", subcore_axis_name="s"),
        compiler_params=pltpu.CompilerParams(needs_layout_passes=False),
        scratch_types=[
            pltpu.VMEM((224,), jnp.float32),
            pltpu.VMEM((_RB * _W,), jnp.float32),
            pltpu.VMEM((_RB * _W,), jnp.float32),
            pltpu.VMEM((16,), jnp.float32),
        ],
    )
    part = call(ypr, yr, tab)
    s = jnp.sum(part, axis=0)
    return (s[0] + s[1] + s[2]) / _B


# speedup vs baseline: 483.9007x; 56.8357x over previous
import jax
import jax.numpy as jnp
from jax import lax
from jax.experimental import pallas as pl
from jax.experimental.pallas import tpu as pltpu
from jax.experimental.pallas import tpu_sc as plsc

_B = 16384
_E = 210
_NC = 2
_NS = 16
_NW = _NC * _NS
_SW = 2 * 128 * 128
_CW = _SW // 2
_HI = 222.0


def _p1_body(zp_hbm, zy_hbm, tab_hbm, rd_hbm, znir_hbm,
             tab_v, bp0, bp1, by0, by1, rd_v, tmp_v,
             sp0, sp1, sy0, sy1):
    w = lax.axis_index("s") * _NC + lax.axis_index("c")
    pltpu.sync_copy(tab_hbm, tab_v)
    zeros = jnp.zeros((16,), jnp.float32)

    def zero_body(i, _):
        rd_v[pl.ds(i * 16, 16)] = zeros
        return 0

    lax.fori_loop(0, _B // 16, zero_body, 0)

    lo = jnp.where(w < 18, 7 * w, 6 * w + 18)
    ns = jnp.where(w < 18, 7, 6)
    base = lo * _SW

    pltpu.make_async_copy(zp_hbm.at[pl.ds(base, _CW)], bp0, sp0).start()
    pltpu.make_async_copy(zy_hbm.at[pl.ds(base, _CW)], by0, sy0).start()
    pltpu.make_async_copy(zp_hbm.at[pl.ds(base + _CW, _CW)], bp1, sp1).start()
    pltpu.make_async_copy(zy_hbm.at[pl.ds(base + _CW, _CW)], by1, sy1).start()

    def compute_chunk(h, bp, by, zn, ir):
        def ibody(i, carry):
            zn, ir = carry
            bt2 = i // 8
            g = i - bt2 * 8
            offi = bt2 * 256 + g * 16
            x = bp[pl.ds(offi, 16)]
            a = bp[pl.ds(offi + 128, 16)]
            ti = by[pl.ds(offi, 16)]
            ta = by[pl.ds(offi + 128, 16)]
            p = jnp.exp(x * -8.0)
            q = jnp.exp(a * -8.0)
            zn = zn + (2.0 * (p + q) - 4.0 * (p * q)) / ((1.0 + p) * (1.0 + q))
            ir = ir + jnp.maximum(x - _HI, 0.0)
            valid = (x > 0.0) & (x <= _HI)
            xi = jnp.where(valid, x, 0.0).astype(jnp.int32)
            delta = (plsc.load_gather(tab_v, [ti.astype(jnp.int32)]) * ta
                     - plsc.load_gather(tab_v, [xi]) * a)
            r0 = h * 8192 + bt2 * 128 + g * 16
            plsc.addupdate(rd_v.at[pl.ds(r0, 16)], delta)
            return zn, ir

        return lax.fori_loop(0, 512, ibody, (zn, ir))

    def pair_body(j, carry):
        zn, ir = carry
        c0 = 2 * j
        more = j + 1 < ns
        pltpu.make_async_copy(zp_hbm.at[pl.ds(0, _CW)], bp0, sp0).wait()
        pltpu.make_async_copy(zy_hbm.at[pl.ds(0, _CW)], by0, sy0).wait()
        zn, ir = compute_chunk(0, bp0, by0, zn, ir)

        @pl.when(more)
        def _():
            off = base + (c0 + 2) * _CW
            pltpu.make_async_copy(zp_hbm.at[pl.ds(off, _CW)], bp0, sp0).start()
            pltpu.make_async_copy(zy_hbm.at[pl.ds(off, _CW)], by0, sy0).start()

        pltpu.make_async_copy(zp_hbm.at[pl.ds(0, _CW)], bp1, sp1).wait()
        pltpu.make_async_copy(zy_hbm.at[pl.ds(0, _CW)], by1, sy1).wait()
        zn, ir = compute_chunk(1, bp1, by1, zn, ir)

        @pl.when(more)
        def _():
            off = base + (c0 + 3) * _CW
            pltpu.make_async_copy(zp_hbm.at[pl.ds(off, _CW)], bp1, sp1).start()
            pltpu.make_async_copy(zy_hbm.at[pl.ds(off, _CW)], by1, sy1).start()

        return zn, ir

    zn, ir = lax.fori_loop(0, ns, pair_body, (zeros, zeros))

    lanes = lax.iota(jnp.int32, 16)
    znS = jnp.sum(zn)
    irS = jnp.sum(ir)
    tmp_v[...] = jnp.where(lanes == 0, znS, jnp.where(lanes == 1, irS, 0.0))
    pltpu.sync_copy(tmp_v, znir_hbm.at[w])
    pltpu.sync_copy(rd_v, rd_hbm.at[w])


def _p2_body(rd_ref, znir_ref, o_ref):
    s = jnp.sum(rd_ref[...], axis=0, keepdims=True)
    val = jnp.sum(s * s) + jnp.sum(znir_ref[...])
    o_ref[...] = jnp.reshape(val, (1, 1))


def kernel(y_pred, y, data):
    zp = y_pred.reshape(128, 128, 7, 3, 10, 2).transpose(2, 3, 4, 0, 5, 1).reshape(-1)
    zy = y.reshape(128, 128, 7, 3, 10, 2).transpose(2, 3, 4, 0, 5, 1).reshape(-1)
    tab = jnp.concatenate([data[:, 0], jnp.zeros((1,), jnp.float32)]) * (1.0 / 700.0)

    p1 = pl.kernel(
        _p1_body,
        out_type=(
            jax.ShapeDtypeStruct((_NW, _B), jnp.float32),
            jax.ShapeDtypeStruct((_NW, 16), jnp.float32),
        ),
        mesh=plsc.VectorSubcoreMesh(core_axis_name="c", subcore_axis_name="s"),
        compiler_params=pltpu.CompilerParams(needs_layout_passes=False),
        scratch_types=[
            pltpu.VMEM((224,), jnp.float32),
            pltpu.VMEM((_CW,), jnp.float32),
            pltpu.VMEM((_CW,), jnp.float32),
            pltpu.VMEM((_CW,), jnp.float32),
            pltpu.VMEM((_CW,), jnp.float32),
            pltpu.VMEM((_B,), jnp.float32),
            pltpu.VMEM((16,), jnp.float32),
            pltpu.SemaphoreType.DMA,
            pltpu.SemaphoreType.DMA,
            pltpu.SemaphoreType.DMA,
            pltpu.SemaphoreType.DMA,
        ],
    )
    rd, znir = p1(zp, zy, tab)

    out2 = pl.pallas_call(
        _p2_body,
        out_shape=jax.ShapeDtypeStruct((1, 1), jnp.float32),
    )(rd, znir)
    return out2[0, 0] / _B


# speedup vs baseline: 813.4202x; 1.6810x over previous
import jax
import jax.numpy as jnp
from jax import lax
from jax.experimental import pallas as pl
from jax.experimental.pallas import tpu as pltpu
from jax.experimental.pallas import tpu_sc as plsc

_B = 16384
_E = 210
_NC = 2
_NS = 16
_NW = _NC * _NS
_SW = 2 * 128 * 128
_CW = _SW // 2
_HI = 222.0


def _p1_body(zp_hbm, zy_hbm, tab_hbm, rd_hbm, znir_hbm,
             tab_v, bp0, bp1, by0, by1, rd_v, tmp_v,
             sp0, sp1, sy0, sy1):
    w = lax.axis_index("s") * _NC + lax.axis_index("c")
    pltpu.sync_copy(tab_hbm, tab_v)
    zeros = jnp.zeros((16,), jnp.float32)

    lo = jnp.where(w < 18, 7 * w, 6 * w + 18)
    ns = jnp.where(w < 18, 7, 6)
    base = lo * _SW

    pltpu.make_async_copy(zp_hbm.at[pl.ds(base, _CW)], bp0, sp0).start()
    pltpu.make_async_copy(zy_hbm.at[pl.ds(base, _CW)], by0, sy0).start()
    pltpu.make_async_copy(zp_hbm.at[pl.ds(base + _CW, _CW)], bp1, sp1).start()
    pltpu.make_async_copy(zy_hbm.at[pl.ds(base + _CW, _CW)], by1, sy1).start()

    def compute_chunk(h, bp, by, zn, ir, first):
        def bt_body(bt2, carry):
            zn, ir = carry
            offb = bt2 * 256
            r0 = h * 8192 + bt2 * 128
            for g in range(8):
                o = offb + g * 16
                x = bp[pl.ds(o, 16)]
                a = bp[pl.ds(o + 128, 16)]
                ti = by[pl.ds(o, 16)]
                ta = by[pl.ds(o + 128, 16)]
                p = jnp.exp(x * -8.0)
                q = jnp.exp(a * -8.0)
                zn = zn + (2.0 * (p + q) - 4.0 * (p * q)) / ((1.0 + p) * (1.0 + q))
                ir = ir + jnp.maximum(x - _HI, 0.0)
                valid = (x > 0.0) & (x <= _HI)
                xi = jnp.where(valid, x, 0.0).astype(jnp.int32)
                delta = (plsc.load_gather(tab_v, [ti.astype(jnp.int32)]) * ta
                         - plsc.load_gather(tab_v, [xi]) * a)
                if first:
                    rd_v[pl.ds(r0 + g * 16, 16)] = delta
                else:
                    plsc.addupdate(rd_v.at[pl.ds(r0 + g * 16, 16)], delta)
            return zn, ir

        return plsc.parallel_loop(0, 64, 1, unroll=2, carry=(zn, ir))(bt_body)

    def pair_iter(c0, zn, ir, more, first):
        pltpu.make_async_copy(zp_hbm.at[pl.ds(0, _CW)], bp0, sp0).wait()
        pltpu.make_async_copy(zy_hbm.at[pl.ds(0, _CW)], by0, sy0).wait()
        zn, ir = compute_chunk(0, bp0, by0, zn, ir, first)

        @pl.when(more)
        def _():
            off = base + (c0 + 2) * _CW
            pltpu.make_async_copy(zp_hbm.at[pl.ds(off, _CW)], bp0, sp0).start()
            pltpu.make_async_copy(zy_hbm.at[pl.ds(off, _CW)], by0, sy0).start()

        pltpu.make_async_copy(zp_hbm.at[pl.ds(0, _CW)], bp1, sp1).wait()
        pltpu.make_async_copy(zy_hbm.at[pl.ds(0, _CW)], by1, sy1).wait()
        zn, ir = compute_chunk(1, bp1, by1, zn, ir, first)

        @pl.when(more)
        def _():
            off = base + (c0 + 3) * _CW
            pltpu.make_async_copy(zp_hbm.at[pl.ds(off, _CW)], bp1, sp1).start()
            pltpu.make_async_copy(zy_hbm.at[pl.ds(off, _CW)], by1, sy1).start()

        return zn, ir

    zn, ir = pair_iter(0, zeros, zeros, 1 < ns, True)

    def pair_body(j, carry):
        zn, ir = carry
        return pair_iter(2 * j, zn, ir, j + 1 < ns, False)

    zn, ir = lax.fori_loop(1, ns, pair_body, (zn, ir))

    lanes = lax.iota(jnp.int32, 16)
    znS = jnp.sum(zn)
    irS = jnp.sum(ir)
    tmp_v[...] = jnp.where(lanes == 0, znS, jnp.where(lanes == 1, irS, 0.0))
    pltpu.sync_copy(tmp_v, znir_hbm.at[w])
    pltpu.sync_copy(rd_v, rd_hbm.at[w])


def _p2_body(rd_ref, znir_ref, o_ref):
    s = jnp.sum(rd_ref[...], axis=0, keepdims=True)
    val = jnp.sum(s * s) + jnp.sum(znir_ref[...])
    o_ref[...] = jnp.reshape(val, (1, 1))


def kernel(y_pred, y, data):
    zp = y_pred.reshape(128, 128, 7, 3, 10, 2).transpose(2, 3, 4, 0, 5, 1).reshape(-1)
    zy = y.reshape(128, 128, 7, 3, 10, 2).transpose(2, 3, 4, 0, 5, 1).reshape(-1)
    tab = jnp.concatenate([data[:, 0], jnp.zeros((1,), jnp.float32)]) * (1.0 / 700.0)

    p1 = pl.kernel(
        _p1_body,
        out_type=(
            jax.ShapeDtypeStruct((_NW, _B), jnp.float32),
            jax.ShapeDtypeStruct((_NW, 16), jnp.float32),
        ),
        mesh=plsc.VectorSubcoreMesh(core_axis_name="c", subcore_axis_name="s"),
        compiler_params=pltpu.CompilerParams(needs_layout_passes=False),
        scratch_types=[
            pltpu.VMEM((224,), jnp.float32),
            pltpu.VMEM((_CW,), jnp.float32),
            pltpu.VMEM((_CW,), jnp.float32),
            pltpu.VMEM((_CW,), jnp.float32),
            pltpu.VMEM((_CW,), jnp.float32),
            pltpu.VMEM((_B,), jnp.float32),
            pltpu.VMEM((16,), jnp.float32),
            pltpu.SemaphoreType.DMA,
            pltpu.SemaphoreType.DMA,
            pltpu.SemaphoreType.DMA,
            pltpu.SemaphoreType.DMA,
        ],
    )
    rd, znir = p1(zp, zy, tab)

    out2 = pl.pallas_call(
        _p2_body,
        out_shape=jax.ShapeDtypeStruct((1, 1), jnp.float32),
    )(rd, znir)
    return out2[0, 0] / _B
